# Initial kernel scaffold; baseline (speedup 1.0000x reference)
import jax
import jax.numpy as jnp
from jax.experimental import pallas as pl


def kernel(x, edge_index, edge_weight, params):
    raise NotImplementedError("write your pallas kernel here")


# speedup vs baseline: 30.3352x; 30.3352x over previous
import jax
import jax.numpy as jnp
from jax import lax
from jax.experimental import pallas as pl
from jax.experimental.pallas import tpu as pltpu
from jax.experimental.pallas import tpu_sc as plsc

N = 5000
NT = 10000
C = 128
E = 320000
H = 64
G = 20000
GP = 20480
DW = 16
ETS = ('s2s', 's2t', 't2s', 't2t')

_NC, _NS = 2, 16
_EPA = E // (_NC * _NS)
_KA = 80
_EPB = E // _NS
_KB = 128
_NFB = _EPB // _KB
_KT = _EPB - _NFB * _KB


def _pre_body(x_ref, wj_ref, wi_ref, w3_ref, wep_ref, bep_ref, batt_ref,
              ls_ref, ld_ref,
              y_ref, z_ref, as_ref, ad_ref, v_ref):
    x = x_ref[...]
    xs = x_ref[pl.ds(0, N), :]
    xt = x_ref[pl.ds(N, N), :]
    as_ref[...] = jnp.dot(x, ls_ref[...], preferred_element_type=jnp.float32)
    ad_ref[...] = jnp.dot(x, ld_ref[...], preferred_element_type=jnp.float32)
    for h in range(2):
        cs = slice(h * H, (h + 1) * H)
        for (tn, dt, t) in ((0, 0, 0), (0, 1, 1), (1, 0, 2), (1, 1, 3)):
            blk = jnp.dot(xs if tn == 0 else xt, wj_ref[t],
                          preferred_element_type=jnp.float32)
            y_ref[h, dt, pl.ds(tn * N, N), :] = blk[:, cs]
        for (tn, st, t) in ((0, 0, 0), (0, 1, 2), (1, 0, 1), (1, 1, 3)):
            c_t = (jnp.dot(bep_ref[pl.ds(t, 1), :], w3_ref[t],
                           preferred_element_type=jnp.float32)
                   + batt_ref[pl.ds(t, 1), :])
            blk = jnp.dot(xs if tn == 0 else xt, wi_ref[t],
                          preferred_element_type=jnp.float32) + c_t
            z_ref[h, st, pl.ds(tn * N, N), :] = blk[:, cs]
        for t in range(4):
            vt = jnp.dot(wep_ref[pl.ds(t, 1), :], w3_ref[t],
                         preferred_element_type=jnp.float32)
            v_ref[h, pl.ds(t, 1), :] = vt[:, cs]


def _tc_precompute(x, wj4, wi4, w34, wep4, bep4, batt4, lsT, ldT):
    f32 = jnp.float32
    return pl.pallas_call(
        _pre_body,
        compiler_params=pltpu.CompilerParams(vmem_limit_bytes=100 * 1024 * 1024),
        out_shape=(
            jax.ShapeDtypeStruct((2, 2, NT, H), f32),
            jax.ShapeDtypeStruct((2, 2, NT, H), f32),
            jax.ShapeDtypeStruct((NT, 4), f32),
            jax.ShapeDtypeStruct((NT, 4), f32),
            jax.ShapeDtypeStruct((2, 4, H), f32),
        ),
    )(x, wj4, wi4, w34, wep4, bep4, batt4, lsT, ldT)


def _pa_body(as_hbm, ad_hbm, src_hbm, dst_hbm, z1_hbm,
             ex_hbm, den_hbm,
             asv, adv, srcv, dstv, izv, exv, exw, densp):
    c = lax.axis_index("c")
    s = lax.axis_index("s")
    wid = s * _NC + c
    pltpu.sync_copy(as_hbm, asv)
    pltpu.sync_copy(ad_hbm, adv)
    pltpu.sync_copy(z1_hbm, densp.at[pl.ds(s * 1280, 1280), :])

    def zrow(i, carry):
        exw[i, :] = jnp.zeros((16,), jnp.float32)
        return carry
    lax.fori_loop(0, _KA, zrow, 0)
    plsc.subcore_barrier()

    zero16 = jnp.zeros((16,), jnp.int32)
    lanes = lax.iota(jnp.int32, 16)

    def chunk(ci, carry):
        eb = wid * _EPA + ci * _KA
        pltpu.sync_copy(src_hbm.at[pl.ds(eb, _KA)], srcv)
        pltpu.sync_copy(dst_hbm.at[pl.ds(eb, _KA)], dstv)
        for j in range(_KA // 16):
            sl = pl.ds(j * 16, 16)
            sv = srcv[sl]
            dv = dstv[sl]
            stv = (sv >= N).astype(jnp.int32)
            dtv = (dv >= N).astype(jnp.int32)
            tv = 2 * stv + dtv
            a = (plsc.load_gather(asv, [sv * 4 + tv])
                 + plsc.load_gather(adv, [dv * 4 + tv]))
            a = jnp.where(a >= 0., a, 0.2 * a)
            e = jnp.exp(a)
            exv[sl] = e
            izv[sl] = stv * NT + dv
            plsc.store_scatter(exw, [j * 16 + lanes, zero16], e)
        pltpu.sync_copy(exv, ex_hbm.at[pl.ds(eb, _KA)])
        pltpu.sync_copy(exw, densp.at[izv], add=True)
        return carry
    lax.fori_loop(0, _EPA // _KA, chunk, 0)
    plsc.subcore_barrier()
    pltpu.sync_copy(densp.at[pl.ds(s * 1280, 1280), :],
                    den_hbm.at[c, pl.ds(s * 1280, 1280), :])


def _sc_pass_a(as_flat, ad_flat, src, dst, z1):
    f32 = jnp.float32
    mesh = plsc.VectorSubcoreMesh(core_axis_name="c", subcore_axis_name="s")
    return pl.kernel(
        _pa_body,
        out_type=(
            jax.ShapeDtypeStruct((E,), f32),
            jax.ShapeDtypeStruct((_NC, GP, DW), f32),
        ),
        mesh=mesh,
        compiler_params=pltpu.CompilerParams(needs_layout_passes=False,
                                             use_tc_tiling_on_sc=False),
        scratch_types=[
            pltpu.VMEM((4 * NT,), f32),
            pltpu.VMEM((4 * NT,), f32),
            pltpu.VMEM((_KA,), jnp.int32),
            pltpu.VMEM((_KA,), jnp.int32),
            pltpu.VMEM((_KA,), jnp.int32),
            pltpu.VMEM((_KA,), f32),
            pltpu.VMEM((_KA, DW), f32),
            pltpu.VMEM_SHARED((GP, DW), f32),
        ],
    )(as_flat, ad_flat, src, dst, z1)


def _pb_body(ytab, ztab, xtab, vtab, src_hbm, dst_hbm, w_hbm, ex_hbm, z2_hbm,
             part_hbm,
             vv, aggsp, sem0, sem1, sem2,
             srcv, dstv, wv, exv, ixv, iyv, izv, izs, tvv, Xv, Yv, Zv, Mv,
             srcv2, dstv2, wv2, exv2, ixv2, iyv2, izv2, izs2, tvv2,
             Xv2, Yv2, Zv2, Mv2):
    c = lax.axis_index("c")
    s = lax.axis_index("s")
    pltpu.sync_copy(vtab.at[c], vv)
    pltpu.sync_copy(z2_hbm, aggsp.at[pl.ds(s * 1280, 1280), :])
    plsc.subcore_barrier()
    base = s * _EPB
    coff1 = c * NT
    coff2 = c * G

    def run(eb, k, srcv, dstv, wv, exv, ixv, iyv, izv, izs, tvv, Xv, Yv, Zv, Mv):
        pltpu.sync_copy(src_hbm.at[pl.ds(eb, k)], srcv)
        pltpu.sync_copy(dst_hbm.at[pl.ds(eb, k)], dstv)
        pltpu.sync_copy(w_hbm.at[pl.ds(eb, k)], wv)
        pltpu.sync_copy(ex_hbm.at[pl.ds(eb, k)], exv)
        for j in range(k // 16):
            sl = pl.ds(j * 16, 16)
            sv = srcv[sl]
            dv = dstv[sl]
            stv = (sv >= N).astype(jnp.int32)
            dtv = (dv >= N).astype(jnp.int32)
            ixv[sl] = coff1 + sv
            iyv[sl] = coff2 + dtv * NT + sv
            izv[sl] = coff2 + stv * NT + dv
            izs[sl] = stv * NT + dv
            tvv[sl] = 2 * stv + dtv
        cp0 = pltpu.async_copy(xtab.at[ixv], Xv, sem0)
        cp1 = pltpu.async_copy(ytab.at[iyv], Yv, sem1)
        cp2 = pltpu.async_copy(ztab.at[izv], Zv, sem2)
        cp0.wait()
        cp1.wait()
        cp2.wait()
        lanes = lax.iota(jnp.int32, 16)

        def edge_group(g, carry):
            tvec = tvv[pl.ds(g * 16, 16)]
            wvec = wv[pl.ds(g * 16, 16)]
            evec = exv[pl.ds(g * 16, 16)]
            for i16 in range(16):
                i = g * 16 + i16
                t_i = tvec[i16]
                w_i = wvec[i16]
                e_i = evec[i16]
                for jj in range(4):
                    slc = pl.ds(jj * 16, 16)
                    vr = plsc.load_gather(vv, [t_i * H + jj * 16 + lanes])
                    Mv[i, slc] = ((e_i * Xv[i, slc])
                                  * (Yv[i, slc] + Zv[i, slc] + w_i * vr))
            return carry
        lax.fori_loop(0, k // 16, edge_group, 0)
        pltpu.sync_copy(Mv, aggsp.at[izs], add=True)

    def ch(ci, carry):
        run(base + ci * _KB, _KB,
            srcv, dstv, wv, exv, ixv, iyv, izv, izs, tvv, Xv, Yv, Zv, Mv)
        return carry
    lax.fori_loop(0, _NFB, ch, 0)
    run(base + _NFB * _KB, _KT,
        srcv2, dstv2, wv2, exv2, ixv2, iyv2, izv2, izs2, tvv2,
        Xv2, Yv2, Zv2, Mv2)
    plsc.subcore_barrier()
    pltpu.sync_copy(aggsp.at[pl.ds(s * 1280, 1280), :],
                    part_hbm.at[c, pl.ds(s * 1280, 1280), :])


def _sc_pass_b(ytab, ztab, xtab, vtab, src, dst, w, ex, z2):
    f32, i32 = jnp.float32, jnp.int32
    mesh = plsc.VectorSubcoreMesh(core_axis_name="c", subcore_axis_name="s")

    def bufs(k):
        return [
            pltpu.VMEM((k,), i32),
            pltpu.VMEM((k,), i32),
            pltpu.VMEM((k,), f32),
            pltpu.VMEM((k,), f32),
            pltpu.VMEM((k,), i32),
            pltpu.VMEM((k,), i32),
            pltpu.VMEM((k,), i32),
            pltpu.VMEM((k,), i32),
            pltpu.VMEM((k,), i32),
            pltpu.VMEM((k, H), f32),
            pltpu.VMEM((k, H), f32),
            pltpu.VMEM((k, H), f32),
            pltpu.VMEM((k, H), f32),
        ]
    return pl.kernel(
        _pb_body,
        out_type=jax.ShapeDtypeStruct((_NC, GP, H), f32),
        mesh=mesh,
        compiler_params=pltpu.CompilerParams(needs_layout_passes=False,
                                             use_tc_tiling_on_sc=False),
        scratch_types=(
            [pltpu.VMEM((4 * H,), f32),
             pltpu.VMEM_SHARED((GP, H), f32),
             pltpu.SemaphoreType.DMA,
             pltpu.SemaphoreType.DMA,
             pltpu.SemaphoreType.DMA]
            + bufs(_KB) + bufs(_KT)
        ),
    )(ytab, ztab, xtab, vtab, src, dst, w, ex, z2)


def _post_body(p_ref, den_ref, x_ref, wagg1_ref, wagg2_ref, bagg_ref,
               wks_ref, bks_ref, qs_ref, wkt_ref, bkt_ref, qt_ref, out_ref):
    outs = []
    for t, (st, dn) in enumerate(((0, 0), (0, 1), (1, 0), (1, 1))):
        r0 = st * NT + dn * N
        ag = jnp.concatenate(
            [p_ref[0, pl.ds(r0, N), :], p_ref[1, pl.ds(r0, N), :]], axis=1)
        den = (den_ref[0, pl.ds(r0, N), 0:1]
               + den_ref[1, pl.ds(r0, N), 0:1] + 1e-16)
        ag = ag / den
        xd = x_ref[pl.ds(dn * N, N), :]
        o = (jnp.dot(ag, wagg1_ref[t], preferred_element_type=jnp.float32)
             + jnp.dot(xd, wagg2_ref[t], preferred_element_type=jnp.float32)
             + bagg_ref[pl.ds(t, 1), :])
        outs.append(jax.nn.relu(o))

    def group(o0, o1, wk_ref, bk_ref, q_ref, row0):
        m0 = jnp.mean(jnp.tanh(
            jnp.dot(o0, wk_ref[...], preferred_element_type=jnp.float32)
            + bk_ref[...]), axis=0, keepdims=True)
        m1 = jnp.mean(jnp.tanh(
            jnp.dot(o1, wk_ref[...], preferred_element_type=jnp.float32)
            + bk_ref[...]), axis=0, keepdims=True)
        s0 = jnp.sum(q_ref[...] * m0, axis=1, keepdims=True)
        s1 = jnp.sum(q_ref[...] * m1, axis=1, keepdims=True)
        sm = jnp.maximum(s0, s1)
        e0 = jnp.exp(s0 - sm)
        e1 = jnp.exp(s1 - sm)
        tot = e0 + e1
        out_ref[pl.ds(row0, N), :] = (e0 / tot) * o0 + (e1 / tot) * o1

    group(outs[0], outs[2], wks_ref, bks_ref, qs_ref, 0)
    group(outs[3], outs[1], wkt_ref, bkt_ref, qt_ref, N)


def _tc_post(p, den, x, wagg1, wagg2, bagg, wks, bks, qs, wkt, bkt, qt):
    return pl.pallas_call(
        _post_body,
        compiler_params=pltpu.CompilerParams(vmem_limit_bytes=100 * 1024 * 1024),
        out_shape=jax.ShapeDtypeStruct((NT, C), jnp.float32),
    )(p, den, x, wagg1, wagg2, bagg, wks, bks, qs, wkt, bkt, qt)


def kernel(x, edge_index, edge_weight, params):
    f32 = jnp.float32
    wj4 = jnp.stack([params['W_att_' + et][0:C] for et in ETS])
    wi4 = jnp.stack([params['W_att_' + et][C:2 * C] for et in ETS])
    w34 = jnp.stack([params['W_att_' + et][2 * C:3 * C] for et in ETS])
    wep4 = jnp.stack([params['W_ep_' + et][0] for et in ETS])
    bep4 = jnp.stack([params['b_ep_' + et] for et in ETS])
    batt4 = jnp.stack([params['b_att_' + et] for et in ETS])
    lsT = jnp.stack([params['lsrc_' + et] for et in ETS], axis=1)
    ldT = jnp.stack([params['ldst_' + et] for et in ETS], axis=1)

    y, z, as_, ad_, v = _tc_precompute(
        x, wj4, wi4, w34, wep4, bep4, batt4, lsT, ldT)
    xh = jnp.stack([x[:, :H], x[:, H:]])

    src = edge_index[0]
    dst = edge_index[1]
    z1 = jnp.zeros((1280, DW), f32)
    ex, den = _sc_pass_a(as_.reshape(-1), ad_.reshape(-1), src, dst, z1)

    z2 = jnp.zeros((1280, H), f32)
    part = _sc_pass_b(y.reshape(2 * G, H), z.reshape(2 * G, H),
                      xh.reshape(2 * NT, H), v.reshape(2, 4 * H),
                      src, dst, edge_weight, ex, z2)

    wagg1 = jnp.stack([params['W_agg_' + et][:C] for et in ETS])
    wagg2 = jnp.stack([params['W_agg_' + et][C:] for et in ETS])
    bagg = jnp.stack([params['b_agg_' + et] for et in ETS])
    return _tc_post(part, den, x, wagg1, wagg2, bagg,
                    params['Wk_s'], params['bk_s'].reshape(1, C), params['q_s'],
                    params['Wk_t'], params['bk_t'].reshape(1, C), params['q_t'])


# speedup vs baseline: 43.8587x; 1.4458x over previous
import jax
import jax.numpy as jnp
from jax import lax
from jax.experimental import pallas as pl
from jax.experimental.pallas import tpu as pltpu
from jax.experimental.pallas import tpu_sc as plsc

N = 5000
NT = 10000
C = 128
E = 320000
H = 64
G = 20000
GP = 20000
DW = 16
ETS = ('s2s', 's2t', 't2s', 't2t')

_NC, _NS = 2, 16
_EPA = E // (_NC * _NS)
_KA = 80
_EPB = E // _NS
_KB = 80
_NFB = _EPB // _KB


def _pre_body(x_ref, wj_ref, wi_ref, w3_ref, wep_ref, bep_ref, batt_ref,
              ls_ref, ld_ref,
              y_ref, z_ref, as_ref, ad_ref, v_ref):
    x = x_ref[...]
    xs = x_ref[pl.ds(0, N), :]
    xt = x_ref[pl.ds(N, N), :]
    as_ref[...] = jnp.dot(x, ls_ref[...], preferred_element_type=jnp.float32)
    ad_ref[...] = jnp.dot(x, ld_ref[...], preferred_element_type=jnp.float32)
    for h in range(2):
        cs = slice(h * H, (h + 1) * H)
        for (tn, dt, t) in ((0, 0, 0), (0, 1, 1), (1, 0, 2), (1, 1, 3)):
            blk = jnp.dot(xs if tn == 0 else xt, wj_ref[t],
                          preferred_element_type=jnp.float32)
            y_ref[h, dt, pl.ds(tn * N, N), :] = blk[:, cs]
        for (tn, st, t) in ((0, 0, 0), (0, 1, 2), (1, 0, 1), (1, 1, 3)):
            c_t = (jnp.dot(bep_ref[pl.ds(t, 1), :], w3_ref[t],
                           preferred_element_type=jnp.float32)
                   + batt_ref[pl.ds(t, 1), :])
            blk = jnp.dot(xs if tn == 0 else xt, wi_ref[t],
                          preferred_element_type=jnp.float32) + c_t
            z_ref[h, st, pl.ds(tn * N, N), :] = blk[:, cs]
        for t in range(4):
            vt = jnp.dot(wep_ref[pl.ds(t, 1), :], w3_ref[t],
                         preferred_element_type=jnp.float32)
            v_ref[h, pl.ds(t, 1), :] = vt[:, cs]


def _tc_precompute(x, wj4, wi4, w34, wep4, bep4, batt4, lsT, ldT):
    f32 = jnp.float32
    return pl.pallas_call(
        _pre_body,
        compiler_params=pltpu.CompilerParams(vmem_limit_bytes=100 * 1024 * 1024),
        out_shape=(
            jax.ShapeDtypeStruct((2, 2, NT, H), f32),
            jax.ShapeDtypeStruct((2, 2, NT, H), f32),
            jax.ShapeDtypeStruct((NT, 4), f32),
            jax.ShapeDtypeStruct((NT, 4), f32),
            jax.ShapeDtypeStruct((2, 4, H), f32),
        ),
    )(x, wj4, wi4, w34, wep4, bep4, batt4, lsT, ldT)


def _pa_body(as_hbm, ad_hbm, src_hbm, dst_hbm, z1_hbm,
             ex_hbm, den_hbm,
             asv, adv, srcv, dstv, izv, exv, exw, densp):
    c = lax.axis_index("c")
    s = lax.axis_index("s")
    wid = s * _NC + c
    pltpu.sync_copy(as_hbm, asv)
    pltpu.sync_copy(ad_hbm, adv)
    pltpu.sync_copy(z1_hbm, densp.at[pl.ds(s * 1250, 1250), :])

    def zrow(i, carry):
        exw[i, :] = jnp.zeros((16,), jnp.float32)
        return carry
    lax.fori_loop(0, _KA, zrow, 0)
    plsc.subcore_barrier()

    zero16 = jnp.zeros((16,), jnp.int32)
    lanes = lax.iota(jnp.int32, 16)

    def chunk(ci, carry):
        eb = wid * _EPA + ci * _KA
        pltpu.sync_copy(src_hbm.at[pl.ds(eb, _KA)], srcv)
        pltpu.sync_copy(dst_hbm.at[pl.ds(eb, _KA)], dstv)
        for j in range(_KA // 16):
            sl = pl.ds(j * 16, 16)
            sv = srcv[sl]
            dv = dstv[sl]
            stv = (sv >= N).astype(jnp.int32)
            dtv = (dv >= N).astype(jnp.int32)
            tv = 2 * stv + dtv
            a = (plsc.load_gather(asv, [sv * 4 + tv])
                 + plsc.load_gather(adv, [dv * 4 + tv]))
            a = jnp.where(a >= 0., a, 0.2 * a)
            e = jnp.exp(a)
            exv[sl] = e
            izv[sl] = stv * NT + dv
            plsc.store_scatter(exw, [j * 16 + lanes, zero16], e)
        pltpu.sync_copy(exv, ex_hbm.at[pl.ds(eb, _KA)])
        pltpu.sync_copy(exw, densp.at[izv], add=True)
        return carry
    lax.fori_loop(0, _EPA // _KA, chunk, 0)
    plsc.subcore_barrier()
    pltpu.sync_copy(densp.at[pl.ds(s * 1250, 1250), :],
                    den_hbm.at[c, pl.ds(s * 1250, 1250), :])


def _sc_pass_a(as_flat, ad_flat, src, dst, z1):
    f32 = jnp.float32
    mesh = plsc.VectorSubcoreMesh(core_axis_name="c", subcore_axis_name="s")
    return pl.kernel(
        _pa_body,
        out_type=(
            jax.ShapeDtypeStruct((E,), f32),
            jax.ShapeDtypeStruct((_NC, GP, DW), f32),
        ),
        mesh=mesh,
        compiler_params=pltpu.CompilerParams(needs_layout_passes=False,
                                             use_tc_tiling_on_sc=False),
        scratch_types=[
            pltpu.VMEM((4 * NT,), f32),
            pltpu.VMEM((4 * NT,), f32),
            pltpu.VMEM((_KA,), jnp.int32),
            pltpu.VMEM((_KA,), jnp.int32),
            pltpu.VMEM((_KA,), jnp.int32),
            pltpu.VMEM((_KA,), f32),
            pltpu.VMEM((_KA, DW), f32),
            pltpu.VMEM_SHARED((GP, DW), f32),
        ],
    )(as_flat, ad_flat, src, dst, z1)


def _pb_body(ytab, ztab, xtab, vtab, src_hbm, dst_hbm, w_hbm, ex_hbm, z2_hbm,
             part_hbm,
             vv, aggsp, *flat):
    c = lax.axis_index("c")
    s = lax.axis_index("s")
    SA, SB = flat[0:16], flat[16:32]
    pltpu.sync_copy(vtab.at[c], vv)
    pltpu.sync_copy(z2_hbm, aggsp.at[pl.ds(s * 1250, 1250), :])
    plsc.subcore_barrier()
    base = s * _EPB
    coff1 = c * NT
    coff2 = c * G
    lanes = lax.iota(jnp.int32, 16)

    def front_lin(ci, S, k):
        (srcv, dstv, wv, exv, ixv, iyv, izv, izs, tvv, Xv, Yv, Zv, Mv,
         semL, semG, semS) = S
        eb = base + ci * _KB
        pltpu.async_copy(src_hbm.at[pl.ds(eb, k)], srcv, semL)
        pltpu.async_copy(dst_hbm.at[pl.ds(eb, k)], dstv, semL)
        pltpu.async_copy(w_hbm.at[pl.ds(eb, k)], wv, semL)
        pltpu.async_copy(ex_hbm.at[pl.ds(eb, k)], exv, semL)

    def front(S, k):
        (srcv, dstv, wv, exv, ixv, iyv, izv, izs, tvv, Xv, Yv, Zv, Mv,
         semL, semG, semS) = S
        for r in (srcv, dstv, wv, exv):
            pltpu.make_async_copy(src_hbm.at[pl.ds(base, k)], r, semL).wait()
        for j in range(k // 16):
            sl = pl.ds(j * 16, 16)
            sv = srcv[sl]
            dv = dstv[sl]
            stv = (sv >= N).astype(jnp.int32)
            dtv = (dv >= N).astype(jnp.int32)
            ixv[sl] = coff1 + sv
            iyv[sl] = coff2 + dtv * NT + sv
            izv[sl] = coff2 + stv * NT + dv
            izs[sl] = stv * NT + dv
            tvv[sl] = 2 * stv + dtv
        pltpu.async_copy(xtab.at[ixv], Xv, semG)
        pltpu.async_copy(ytab.at[iyv], Yv, semG)
        pltpu.async_copy(ztab.at[izv], Zv, semG)

    def wait_gath(S):
        (srcv, dstv, wv, exv, ixv, iyv, izv, izs, tvv, Xv, Yv, Zv, Mv,
         semL, semG, semS) = S
        pltpu.make_async_copy(xtab.at[ixv], Xv, semG).wait()
        pltpu.make_async_copy(ytab.at[iyv], Yv, semG).wait()
        pltpu.make_async_copy(ztab.at[izv], Zv, semG).wait()

    def compute(S, k):
        (srcv, dstv, wv, exv, ixv, iyv, izv, izs, tvv, Xv, Yv, Zv, Mv,
         semL, semG, semS) = S

        def grp(g, carry):
            tvec = tvv[pl.ds(g * 16, 16)]
            wvec = wv[pl.ds(g * 16, 16)]
            evec = exv[pl.ds(g * 16, 16)]
            bvec = evec * wvec
            for i16 in range(16):
                i = g * 16 + i16
                t_i = tvec[i16]
                a_i = evec[i16]
                b_i = bvec[i16]
                for jj in range(4):
                    slc = pl.ds(jj * 16, 16)
                    vr = plsc.load_gather(vv, [t_i * H + jj * 16 + lanes])
                    Mv[i, slc] = (Xv[i, slc]
                                  * (a_i * (Yv[i, slc] + Zv[i, slc]) + b_i * vr))
            return carry
        lax.fori_loop(0, k // 16, grp, 0)

    def scat(S):
        (srcv, dstv, wv, exv, ixv, iyv, izv, izs, tvv, Xv, Yv, Zv, Mv,
         semL, semG, semS) = S
        pltpu.async_copy(Mv, aggsp.at[izs], semS, add=True)

    def wait_scat(S):
        (srcv, dstv, wv, exv, ixv, iyv, izv, izs, tvv, Xv, Yv, Zv, Mv,
         semL, semG, semS) = S
        pltpu.make_async_copy(Mv, aggsp.at[izs], semS).wait()

    front_lin(0, SA, _KB)
    front(SA, _KB)
    front_lin(1, SB, _KB)
    pltpu.async_copy(z2_hbm.at[pl.ds(0, _KB), :], SB[12], SB[15])

    def pairs(ci2, carry):
        ca = 2 * ci2
        wait_gath(SA)
        wait_scat(SB)
        front(SB, _KB)
        compute(SA, _KB)
        scat(SA)
        front_lin(ca + 2, SA, _KB)
        wait_gath(SB)
        wait_scat(SA)
        front(SA, _KB)
        compute(SB, _KB)
        scat(SB)
        front_lin(ca + 3, SB, _KB)
        return carry
    lax.fori_loop(0, (_NFB - 2) // 2, pairs, 0)
    wait_gath(SA)
    wait_scat(SB)
    front(SB, _KB)
    compute(SA, _KB)
    scat(SA)
    wait_gath(SB)
    wait_scat(SA)
    compute(SB, _KB)
    scat(SB)
    wait_scat(SB)
    plsc.subcore_barrier()
    pltpu.sync_copy(aggsp.at[pl.ds(s * 1250, 1250), :],
                    part_hbm.at[c, pl.ds(s * 1250, 1250), :])


def _sc_pass_b(ytab, ztab, xtab, vtab, src, dst, w, ex, z2):
    f32, i32 = jnp.float32, jnp.int32
    mesh = plsc.VectorSubcoreMesh(core_axis_name="c", subcore_axis_name="s")

    def bufs(k):
        return [
            pltpu.VMEM((k,), i32),
            pltpu.VMEM((k,), i32),
            pltpu.VMEM((k,), f32),
            pltpu.VMEM((k,), f32),
            pltpu.VMEM((k,), i32),
            pltpu.VMEM((k,), i32),
            pltpu.VMEM((k,), i32),
            pltpu.VMEM((k,), i32),
            pltpu.VMEM((k,), i32),
            pltpu.VMEM((k, H), f32),
            pltpu.VMEM((k, H), f32),
            pltpu.VMEM((k, H), f32),
            pltpu.VMEM((k, H), f32),
            pltpu.SemaphoreType.DMA,
            pltpu.SemaphoreType.DMA,
            pltpu.SemaphoreType.DMA,
        ]
    return pl.kernel(
        _pb_body,
        out_type=jax.ShapeDtypeStruct((_NC, GP, H), f32),
        mesh=mesh,
        compiler_params=pltpu.CompilerParams(needs_layout_passes=False,
                                             use_tc_tiling_on_sc=False),
        scratch_types=(
            [pltpu.VMEM((4 * H,), f32),
             pltpu.VMEM_SHARED((GP, H), f32)]
            + bufs(_KB) + bufs(_KB)
        ),
    )(ytab, ztab, xtab, vtab, src, dst, w, ex, z2)


def _post_body(p_ref, den_ref, x_ref, wagg1_ref, wagg2_ref, bagg_ref,
               wks_ref, bks_ref, qs_ref, wkt_ref, bkt_ref, qt_ref, out_ref):
    outs = []
    for t, (st, dn) in enumerate(((0, 0), (0, 1), (1, 0), (1, 1))):
        r0 = st * NT + dn * N
        ag = jnp.concatenate(
            [p_ref[0, pl.ds(r0, N), :], p_ref[1, pl.ds(r0, N), :]], axis=1)
        den = (den_ref[0, pl.ds(r0, N), 0:1]
               + den_ref[1, pl.ds(r0, N), 0:1] + 1e-16)
        ag = ag / den
        xd = x_ref[pl.ds(dn * N, N), :]
        o = (jnp.dot(ag, wagg1_ref[t], preferred_element_type=jnp.float32)
             + jnp.dot(xd, wagg2_ref[t], preferred_element_type=jnp.float32)
             + bagg_ref[pl.ds(t, 1), :])
        outs.append(jax.nn.relu(o))

    def group(o0, o1, wk_ref, bk_ref, q_ref, row0):
        m0 = jnp.mean(jnp.tanh(
            jnp.dot(o0, wk_ref[...], preferred_element_type=jnp.float32)
            + bk_ref[...]), axis=0, keepdims=True)
        m1 = jnp.mean(jnp.tanh(
            jnp.dot(o1, wk_ref[...], preferred_element_type=jnp.float32)
            + bk_ref[...]), axis=0, keepdims=True)
        s0 = jnp.sum(q_ref[...] * m0, axis=1, keepdims=True)
        s1 = jnp.sum(q_ref[...] * m1, axis=1, keepdims=True)
        sm = jnp.maximum(s0, s1)
        e0 = jnp.exp(s0 - sm)
        e1 = jnp.exp(s1 - sm)
        tot = e0 + e1
        out_ref[pl.ds(row0, N), :] = (e0 / tot) * o0 + (e1 / tot) * o1

    group(outs[0], outs[2], wks_ref, bks_ref, qs_ref, 0)
    group(outs[3], outs[1], wkt_ref, bkt_ref, qt_ref, N)


def _tc_post(p, den, x, wagg1, wagg2, bagg, wks, bks, qs, wkt, bkt, qt):
    return pl.pallas_call(
        _post_body,
        compiler_params=pltpu.CompilerParams(vmem_limit_bytes=100 * 1024 * 1024),
        out_shape=jax.ShapeDtypeStruct((NT, C), jnp.float32),
    )(p, den, x, wagg1, wagg2, bagg, wks, bks, qs, wkt, bkt, qt)


def kernel(x, edge_index, edge_weight, params):
    f32 = jnp.float32
    wj4 = jnp.stack([params['W_att_' + et][0:C] for et in ETS])
    wi4 = jnp.stack([params['W_att_' + et][C:2 * C] for et in ETS])
    w34 = jnp.stack([params['W_att_' + et][2 * C:3 * C] for et in ETS])
    wep4 = jnp.stack([params['W_ep_' + et][0] for et in ETS])
    bep4 = jnp.stack([params['b_ep_' + et] for et in ETS])
    batt4 = jnp.stack([params['b_att_' + et] for et in ETS])
    lsT = jnp.stack([params['lsrc_' + et] for et in ETS], axis=1)
    ldT = jnp.stack([params['ldst_' + et] for et in ETS], axis=1)

    y, z, as_, ad_, v = _tc_precompute(
        x, wj4, wi4, w34, wep4, bep4, batt4, lsT, ldT)
    xh = jnp.stack([x[:, :H], x[:, H:]])

    src = edge_index[0]
    dst = edge_index[1]
    z1 = jnp.zeros((1250, DW), f32)
    ex, den = _sc_pass_a(as_.reshape(-1), ad_.reshape(-1), src, dst, z1)

    z2 = jnp.zeros((1250, H), f32)
    part = _sc_pass_b(y.reshape(2 * G, H), z.reshape(2 * G, H),
                      xh.reshape(2 * NT, H), v.reshape(2, 4 * H),
                      src, dst, edge_weight, ex, z2)

    wagg1 = jnp.stack([params['W_agg_' + et][:C] for et in ETS])
    wagg2 = jnp.stack([params['W_agg_' + et][C:] for et in ETS])
    bagg = jnp.stack([params['b_agg_' + et] for et in ETS])
    return _tc_post(part, den, x, wagg1, wagg2, bagg,
                    params['Wk_s'], params['bk_s'].reshape(1, C), params['q_s'],
                    params['Wk_t'], params['bk_t'].reshape(1, C), params['q_t'])


# speedup vs baseline: 46.6696x; 1.0641x over previous
import jax
import jax.numpy as jnp
from jax import lax
from jax.experimental import pallas as pl
from jax.experimental.pallas import tpu as pltpu
from jax.experimental.pallas import tpu_sc as plsc

N = 5000
NT = 10000
C = 128
E = 320000
H = 64
G = 20000
GP = 20000
DW = 16
ETS = ('s2s', 's2t', 't2s', 't2t')

_NC, _NS = 2, 16
_EPA = E // (_NC * _NS)
_KA = 80
_EPB = E // _NS
_KB = 80
_NFB = _EPB // _KB


def _pre_body(x_ref, wj_ref, wi_ref, w3_ref, wep_ref, bep_ref, batt_ref,
              ls_ref, ld_ref,
              y_ref, z_ref, as_ref, ad_ref, v_ref):
    x = x_ref[...]
    xs = x_ref[pl.ds(0, N), :]
    xt = x_ref[pl.ds(N, N), :]
    as_ref[...] = jnp.dot(x, ls_ref[...], preferred_element_type=jnp.float32)
    ad_ref[...] = jnp.dot(x, ld_ref[...], preferred_element_type=jnp.float32)
    for h in range(2):
        cs = slice(h * H, (h + 1) * H)
        for (tn, dt, t) in ((0, 0, 0), (0, 1, 1), (1, 0, 2), (1, 1, 3)):
            blk = jnp.dot(xs if tn == 0 else xt, wj_ref[t],
                          preferred_element_type=jnp.float32)
            y_ref[h, dt, pl.ds(tn * N, N), :] = blk[:, cs]
        for (tn, st, t) in ((0, 0, 0), (0, 1, 2), (1, 0, 1), (1, 1, 3)):
            c_t = (jnp.dot(bep_ref[pl.ds(t, 1), :], w3_ref[t],
                           preferred_element_type=jnp.float32)
                   + batt_ref[pl.ds(t, 1), :])
            blk = jnp.dot(xs if tn == 0 else xt, wi_ref[t],
                          preferred_element_type=jnp.float32) + c_t
            z_ref[h, st, pl.ds(tn * N, N), :] = blk[:, cs]
        for t in range(4):
            vt = jnp.dot(wep_ref[pl.ds(t, 1), :], w3_ref[t],
                         preferred_element_type=jnp.float32)
            v_ref[h, pl.ds(t, 1), :] = vt[:, cs]


def _tc_precompute(x, wj4, wi4, w34, wep4, bep4, batt4, lsT, ldT):
    f32 = jnp.float32
    return pl.pallas_call(
        _pre_body,
        compiler_params=pltpu.CompilerParams(vmem_limit_bytes=100 * 1024 * 1024),
        out_shape=(
            jax.ShapeDtypeStruct((2, 2, NT, H), f32),
            jax.ShapeDtypeStruct((2, 2, NT, H), f32),
            jax.ShapeDtypeStruct((NT, 4), f32),
            jax.ShapeDtypeStruct((NT, 4), f32),
            jax.ShapeDtypeStruct((2, 4, H), f32),
        ),
    )(x, wj4, wi4, w34, wep4, bep4, batt4, lsT, ldT)


def _pa_body(as_hbm, ad_hbm, src_hbm, dst_hbm, z1_hbm,
             ex_hbm, den_hbm,
             asv, adv, srcv, dstv, izv2, exv2, dummy, densp, semS):
    c = lax.axis_index("c")
    s = lax.axis_index("s")
    wid = s * _NC + c
    pltpu.sync_copy(as_hbm, asv)
    pltpu.sync_copy(ad_hbm, adv)
    pltpu.sync_copy(src_hbm.at[pl.ds(wid * _EPA, _EPA)], srcv)
    pltpu.sync_copy(dst_hbm.at[pl.ds(wid * _EPA, _EPA)], dstv)
    pltpu.sync_copy(z1_hbm, densp.at[pl.ds(s * 1280, 1280)])
    plsc.subcore_barrier()

    pltpu.async_copy(z1_hbm.at[pl.ds(0, _KA)], dummy, semS)

    def row(r, carry):
        for j in range(_KA // 16):
            sl = pl.ds(j * 16, 16)
            base = r * _KA + j * 16
            sv = srcv[pl.ds(base, 16)]
            dv = dstv[pl.ds(base, 16)]
            stv = (sv >= N).astype(jnp.int32)
            dtv = (dv >= N).astype(jnp.int32)
            tv = 2 * stv + dtv
            a = (plsc.load_gather(asv, [sv * 4 + tv])
                 + plsc.load_gather(adv, [dv * 4 + tv]))
            a = jnp.where(a >= 0., a, 0.2 * a)
            exv2[r, sl] = jnp.exp(a)
            izv2[r, sl] = stv * NT + dv
        pltpu.make_async_copy(z1_hbm.at[pl.ds(0, _KA)], dummy, semS).wait()
        pltpu.async_copy(exv2.at[r], densp.at[izv2.at[r]], semS, add=True)
        return carry
    lax.fori_loop(0, _EPA // _KA, row, 0)
    pltpu.make_async_copy(z1_hbm.at[pl.ds(0, _KA)], dummy, semS).wait()
    pltpu.sync_copy(exv2, ex_hbm.at[wid])
    plsc.subcore_barrier()
    pltpu.sync_copy(densp.at[pl.ds(s * 1280, 1280)],
                    den_hbm.at[c, pl.ds(s * 1280, 1280)])


def _sc_pass_a(as_flat, ad_flat, src, dst, z1):
    f32 = jnp.float32
    mesh = plsc.VectorSubcoreMesh(core_axis_name="c", subcore_axis_name="s")
    nr = _EPA // _KA
    return pl.kernel(
        _pa_body,
        out_type=(
            jax.ShapeDtypeStruct((_NC * _NS, nr, _KA), f32),
            jax.ShapeDtypeStruct((_NC, 20480), f32),
        ),
        mesh=mesh,
        compiler_params=pltpu.CompilerParams(needs_layout_passes=False,
                                             use_tc_tiling_on_sc=False),
        scratch_types=[
            pltpu.VMEM((4 * NT,), f32),
            pltpu.VMEM((4 * NT,), f32),
            pltpu.VMEM((_EPA,), jnp.int32),
            pltpu.VMEM((_EPA,), jnp.int32),
            pltpu.VMEM((nr, _KA), jnp.int32),
            pltpu.VMEM((nr, _KA), f32),
            pltpu.VMEM((_KA,), f32),
            pltpu.VMEM_SHARED((20480,), f32),
            pltpu.SemaphoreType.DMA,
        ],
    )(as_flat, ad_flat, src, dst, z1)


def _pb_body(ytab, ztab, xtab, vtab, sd_hbm, we_hbm, z2_hbm,
             part_hbm,
             vv, aggsp, *flat):
    c = lax.axis_index("c")
    s = lax.axis_index("s")
    SA, SB = flat[0:14], flat[14:28]
    pltpu.sync_copy(vtab.at[c], vv)
    pltpu.sync_copy(z2_hbm, aggsp.at[pl.ds(s * 1250, 1250), :])
    plsc.subcore_barrier()
    base = s * _EPB
    coff1 = c * NT
    coff2 = c * G
    lanes = lax.iota(jnp.int32, 16)

    def front_lin(ci, S, k):
        (sdv, wev, ixv, iyv, izv, izs, tvv, Xv, Yv, Zv, Mv,
         semL, semG, semS) = S
        eb = base + ci * _KB
        pltpu.async_copy(sd_hbm.at[pl.ds(2 * eb, 2 * k)], sdv, semL)
        pltpu.async_copy(we_hbm.at[pl.ds(2 * eb, 2 * k)], wev, semL)

    def front(S, k):
        (sdv, wev, ixv, iyv, izv, izs, tvv, Xv, Yv, Zv, Mv,
         semL, semG, semS) = S
        for r in (sdv, wev):
            pltpu.make_async_copy(sd_hbm.at[pl.ds(base, 2 * k)], r, semL).wait()
        for j in range(k // 16):
            sl = pl.ds(j * 16, 16)
            sv = plsc.load_gather(sdv, [j * 32 + 2 * lanes])
            dv = plsc.load_gather(sdv, [j * 32 + 2 * lanes + 1])
            stv = (sv >= N).astype(jnp.int32)
            dtv = (dv >= N).astype(jnp.int32)
            ixv[sl] = coff1 + sv
            iyv[sl] = coff2 + dtv * NT + sv
            izv[sl] = coff2 + stv * NT + dv
            izs[sl] = stv * NT + dv
            tvv[sl] = 2 * stv + dtv
        pltpu.async_copy(xtab.at[ixv], Xv, semG)
        pltpu.async_copy(ytab.at[iyv], Yv, semG)
        pltpu.async_copy(ztab.at[izv], Zv, semG)

    def wait_gath(S):
        (sdv, wev, ixv, iyv, izv, izs, tvv, Xv, Yv, Zv, Mv,
         semL, semG, semS) = S
        pltpu.make_async_copy(xtab.at[ixv], Xv, semG).wait()
        pltpu.make_async_copy(ytab.at[iyv], Yv, semG).wait()
        pltpu.make_async_copy(ztab.at[izv], Zv, semG).wait()

    vre = [vv[pl.ds(t * H + jj * 16, 16)] for t in range(4) for jj in range(4)]

    def compute(S, k):
        (sdv, wev, ixv, iyv, izv, izs, tvv, Xv, Yv, Zv, Mv,
         semL, semG, semS) = S

        def grp(g, carry):
            tvec = tvv[pl.ds(g * 16, 16)]
            wvec = plsc.load_gather(wev, [g * 32 + 2 * lanes])
            evec = plsc.load_gather(wev, [g * 32 + 2 * lanes + 1])
            bvec = evec * wvec
            for i16 in range(16):
                i = g * 16 + i16
                t_i = tvec[i16]
                a_i = evec[i16]
                b_i = bvec[i16]
                is0 = t_i == 0
                is2 = t_i == 2
                lo = t_i < 2
                for jj in range(4):
                    slc = pl.ds(jj * 16, 16)
                    v01 = jnp.where(is0, vre[jj], vre[4 + jj])
                    v23 = jnp.where(is2, vre[8 + jj], vre[12 + jj])
                    vr = jnp.where(lo, v01, v23)
                    Mv[i, slc] = (Xv[i, slc]
                                  * (a_i * (Yv[i, slc] + Zv[i, slc]) + b_i * vr))
            return carry
        lax.fori_loop(0, k // 16, grp, 0)

    def scat(S):
        (sdv, wev, ixv, iyv, izv, izs, tvv, Xv, Yv, Zv, Mv,
         semL, semG, semS) = S
        pltpu.async_copy(Mv, aggsp.at[izs], semS, add=True)

    def wait_scat(S):
        (sdv, wev, ixv, iyv, izv, izs, tvv, Xv, Yv, Zv, Mv,
         semL, semG, semS) = S
        pltpu.make_async_copy(Mv, aggsp.at[izs], semS).wait()

    front_lin(0, SA, _KB)
    front(SA, _KB)
    front_lin(1, SB, _KB)
    pltpu.async_copy(z2_hbm.at[pl.ds(0, _KB), :], SB[10], SB[13])

    def pairs(ci2, carry):
        ca = 2 * ci2
        wait_gath(SA)
        wait_scat(SB)
        front(SB, _KB)
        compute(SA, _KB)
        scat(SA)
        front_lin(ca + 2, SA, _KB)
        wait_gath(SB)
        wait_scat(SA)
        front(SA, _KB)
        compute(SB, _KB)
        scat(SB)
        front_lin(ca + 3, SB, _KB)
        return carry
    lax.fori_loop(0, (_NFB - 2) // 2, pairs, 0)
    wait_gath(SA)
    wait_scat(SB)
    front(SB, _KB)
    compute(SA, _KB)
    scat(SA)
    wait_gath(SB)
    wait_scat(SA)
    compute(SB, _KB)
    scat(SB)
    wait_scat(SB)
    plsc.subcore_barrier()
    pltpu.sync_copy(aggsp.at[pl.ds(s * 1250, 1250), :],
                    part_hbm.at[c, pl.ds(s * 1250, 1250), :])


def _sc_pass_b(ytab, ztab, xtab, vtab, sd, we, z2):
    f32, i32 = jnp.float32, jnp.int32
    mesh = plsc.VectorSubcoreMesh(core_axis_name="c", subcore_axis_name="s")

    def bufs(k):
        return [
            pltpu.VMEM((2 * k,), i32),
            pltpu.VMEM((2 * k,), f32),
            pltpu.VMEM((k,), i32),
            pltpu.VMEM((k,), i32),
            pltpu.VMEM((k,), i32),
            pltpu.VMEM((k,), i32),
            pltpu.VMEM((k,), i32),
            pltpu.VMEM((k, H), f32),
            pltpu.VMEM((k, H), f32),
            pltpu.VMEM((k, H), f32),
            pltpu.VMEM((k, H), f32),
            pltpu.SemaphoreType.DMA,
            pltpu.SemaphoreType.DMA,
            pltpu.SemaphoreType.DMA,
        ]
    return pl.kernel(
        _pb_body,
        out_type=jax.ShapeDtypeStruct((_NC, GP, H), f32),
        mesh=mesh,
        compiler_params=pltpu.CompilerParams(needs_layout_passes=False,
                                             use_tc_tiling_on_sc=False),
        scratch_types=(
            [pltpu.VMEM((4 * H,), f32),
             pltpu.VMEM_SHARED((GP, H), f32)]
            + bufs(_KB) + bufs(_KB)
        ),
    )(ytab, ztab, xtab, vtab, sd, we, z2)


def _post_body(p_ref, den_ref, x_ref, wagg1_ref, wagg2_ref, bagg_ref,
               wks_ref, bks_ref, qs_ref, wkt_ref, bkt_ref, qt_ref, out_ref):
    outs = []
    for t, (st, dn) in enumerate(((0, 0), (0, 1), (1, 0), (1, 1))):
        r0 = st * NT + dn * N
        ag = jnp.concatenate(
            [p_ref[0, pl.ds(r0, N), :], p_ref[1, pl.ds(r0, N), :]], axis=1)
        den = (den_ref[0, pl.ds(r0, N), 0:1]
               + den_ref[1, pl.ds(r0, N), 0:1] + 1e-16)
        ag = ag / den
        xd = x_ref[pl.ds(dn * N, N), :]
        o = (jnp.dot(ag, wagg1_ref[t], preferred_element_type=jnp.float32)
             + jnp.dot(xd, wagg2_ref[t], preferred_element_type=jnp.float32)
             + bagg_ref[pl.ds(t, 1), :])
        outs.append(jax.nn.relu(o))

    def group(o0, o1, wk_ref, bk_ref, q_ref, row0):
        m0 = jnp.mean(jnp.tanh(
            jnp.dot(o0, wk_ref[...], preferred_element_type=jnp.float32)
            + bk_ref[...]), axis=0, keepdims=True)
        m1 = jnp.mean(jnp.tanh(
            jnp.dot(o1, wk_ref[...], preferred_element_type=jnp.float32)
            + bk_ref[...]), axis=0, keepdims=True)
        s0 = jnp.sum(q_ref[...] * m0, axis=1, keepdims=True)
        s1 = jnp.sum(q_ref[...] * m1, axis=1, keepdims=True)
        sm = jnp.maximum(s0, s1)
        e0 = jnp.exp(s0 - sm)
        e1 = jnp.exp(s1 - sm)
        tot = e0 + e1
        out_ref[pl.ds(row0, N), :] = (e0 / tot) * o0 + (e1 / tot) * o1

    group(outs[0], outs[2], wks_ref, bks_ref, qs_ref, 0)
    group(outs[3], outs[1], wkt_ref, bkt_ref, qt_ref, N)


def _tc_post(p, den, x, wagg1, wagg2, bagg, wks, bks, qs, wkt, bkt, qt):
    return pl.pallas_call(
        _post_body,
        compiler_params=pltpu.CompilerParams(vmem_limit_bytes=100 * 1024 * 1024),
        out_shape=jax.ShapeDtypeStruct((NT, C), jnp.float32),
    )(p, den, x, wagg1, wagg2, bagg, wks, bks, qs, wkt, bkt, qt)


def kernel(x, edge_index, edge_weight, params):
    f32 = jnp.float32
    wj4 = jnp.stack([params['W_att_' + et][0:C] for et in ETS])
    wi4 = jnp.stack([params['W_att_' + et][C:2 * C] for et in ETS])
    w34 = jnp.stack([params['W_att_' + et][2 * C:3 * C] for et in ETS])
    wep4 = jnp.stack([params['W_ep_' + et][0] for et in ETS])
    bep4 = jnp.stack([params['b_ep_' + et] for et in ETS])
    batt4 = jnp.stack([params['b_att_' + et] for et in ETS])
    lsT = jnp.stack([params['lsrc_' + et] for et in ETS], axis=1)
    ldT = jnp.stack([params['ldst_' + et] for et in ETS], axis=1)

    y, z, as_, ad_, v = _tc_precompute(
        x, wj4, wi4, w34, wep4, bep4, batt4, lsT, ldT)
    xh = jnp.stack([x[:, :H], x[:, H:]])

    src = edge_index[0]
    dst = edge_index[1]
    z1 = jnp.zeros((1280,), f32)
    ex, den = _sc_pass_a(as_.reshape(-1), ad_.reshape(-1), src, dst, z1)
    ex = ex.reshape(E)
    den = den.reshape(_NC, 20480, 1)

    z2 = jnp.zeros((1250, H), f32)
    sd = jnp.stack([src, dst], axis=1).reshape(2 * E)
    we = jnp.stack([edge_weight, ex], axis=1).reshape(2 * E)
    part = _sc_pass_b(y.reshape(2 * G, H), z.reshape(2 * G, H),
                      xh.reshape(2 * NT, H), v.reshape(2, 4 * H),
                      sd, we, z2)

    wagg1 = jnp.stack([params['W_agg_' + et][:C] for et in ETS])
    wagg2 = jnp.stack([params['W_agg_' + et][C:] for et in ETS])
    bagg = jnp.stack([params['b_agg_' + et] for et in ETS])
    return _tc_post(part, den, x, wagg1, wagg2, bagg,
                    params['Wk_s'], params['bk_s'].reshape(1, C), params['q_s'],
                    params['Wk_t'], params['bk_t'].reshape(1, C), params['q_t'])


# speedup vs baseline: 72.5765x; 1.5551x over previous
import jax
import jax.numpy as jnp
from jax import lax
from jax.experimental import pallas as pl
from jax.experimental.pallas import tpu as pltpu
from jax.experimental.pallas import tpu_sc as plsc

N = 5000
NT = 10000
C = 128
E = 320000
H = 64
G = 20000
GP = 20000
DW = 16
ETS = ('s2s', 's2t', 't2s', 't2t')

_NC, _NS = 2, 16
_EPA = E // (_NC * _NS)
_KA = 80
_EPB = E // _NS
_KB = 80
_NFB = _EPB // _KB


def _pre_body(x_ref, wj_ref, wi_ref, w3_ref, wep_ref, bep_ref, batt_ref,
              ls_ref, ld_ref,
              y_ref, z_ref, as_ref, ad_ref, v_ref):
    x = x_ref[...]
    xs = x_ref[pl.ds(0, N), :]
    xt = x_ref[pl.ds(N, N), :]
    as_ref[...] = jnp.dot(x, ls_ref[...], preferred_element_type=jnp.float32)
    ad_ref[...] = jnp.dot(x, ld_ref[...], preferred_element_type=jnp.float32)
    for h in range(2):
        cs = slice(h * H, (h + 1) * H)
        for (tn, dt, t) in ((0, 0, 0), (0, 1, 1), (1, 0, 2), (1, 1, 3)):
            blk = jnp.dot(xs if tn == 0 else xt, wj_ref[t],
                          preferred_element_type=jnp.float32)
            y_ref[h, dt, pl.ds(tn * N, N), :] = blk[:, cs]
        for (tn, st, t) in ((0, 0, 0), (0, 1, 2), (1, 0, 1), (1, 1, 3)):
            c_t = (jnp.dot(bep_ref[pl.ds(t, 1), :], w3_ref[t],
                           preferred_element_type=jnp.float32)
                   + batt_ref[pl.ds(t, 1), :])
            blk = jnp.dot(xs if tn == 0 else xt, wi_ref[t],
                          preferred_element_type=jnp.float32) + c_t
            z_ref[h, st, pl.ds(tn * N, N), :] = blk[:, cs]
        for t in range(4):
            vt = jnp.dot(wep_ref[pl.ds(t, 1), :], w3_ref[t],
                         preferred_element_type=jnp.float32)
            v_ref[h, pl.ds(t, 1), :] = vt[:, cs]


def _tc_precompute(x, wj4, wi4, w34, wep4, bep4, batt4, lsT, ldT):
    f32 = jnp.float32
    return pl.pallas_call(
        _pre_body,
        compiler_params=pltpu.CompilerParams(vmem_limit_bytes=100 * 1024 * 1024),
        out_shape=(
            jax.ShapeDtypeStruct((2, 2, NT, H), f32),
            jax.ShapeDtypeStruct((2, 2, NT, H), f32),
            jax.ShapeDtypeStruct((NT, 4), f32),
            jax.ShapeDtypeStruct((NT, 4), f32),
            jax.ShapeDtypeStruct((2, 4, H), f32),
        ),
    )(x, wj4, wi4, w34, wep4, bep4, batt4, lsT, ldT)


def _pa_body(as_hbm, ad_hbm, src_hbm, dst_hbm, z1_hbm,
             ex_hbm, den_hbm,
             asv, adv, srcv, dstv, izv2, exv2, dummy, densp, semS):
    c = lax.axis_index("c")
    s = lax.axis_index("s")
    wid = s * _NC + c
    pltpu.sync_copy(as_hbm, asv)
    pltpu.sync_copy(ad_hbm, adv)
    pltpu.sync_copy(src_hbm.at[pl.ds(wid * _EPA, _EPA)], srcv)
    pltpu.sync_copy(dst_hbm.at[pl.ds(wid * _EPA, _EPA)], dstv)
    pltpu.sync_copy(z1_hbm, densp.at[pl.ds(s * 1280, 1280)])
    plsc.subcore_barrier()

    pltpu.async_copy(z1_hbm.at[pl.ds(0, _KA)], dummy, semS)

    def row(r, carry):
        for j in range(_KA // 16):
            sl = pl.ds(j * 16, 16)
            base = r * _KA + j * 16
            sv = srcv[pl.ds(base, 16)]
            dv = dstv[pl.ds(base, 16)]
            stv = (sv >= N).astype(jnp.int32)
            dtv = (dv >= N).astype(jnp.int32)
            tv = 2 * stv + dtv
            a = (plsc.load_gather(asv, [sv * 4 + tv])
                 + plsc.load_gather(adv, [dv * 4 + tv]))
            a = jnp.where(a >= 0., a, 0.2 * a)
            exv2[r, sl] = jnp.exp(a)
            izv2[r, sl] = stv * NT + dv
        pltpu.make_async_copy(z1_hbm.at[pl.ds(0, _KA)], dummy, semS).wait()
        pltpu.async_copy(exv2.at[r], densp.at[izv2.at[r]], semS, add=True)
        return carry
    lax.fori_loop(0, _EPA // _KA, row, 0)
    pltpu.make_async_copy(z1_hbm.at[pl.ds(0, _KA)], dummy, semS).wait()
    pltpu.sync_copy(exv2, ex_hbm.at[wid])
    plsc.subcore_barrier()
    pltpu.sync_copy(densp.at[pl.ds(s * 1280, 1280)],
                    den_hbm.at[c, pl.ds(s * 1280, 1280)])


def _sc_pass_a(as_flat, ad_flat, src, dst, z1):
    f32 = jnp.float32
    mesh = plsc.VectorSubcoreMesh(core_axis_name="c", subcore_axis_name="s")
    nr = _EPA // _KA
    return pl.kernel(
        _pa_body,
        out_type=(
            jax.ShapeDtypeStruct((_NC * _NS, nr, _KA), f32),
            jax.ShapeDtypeStruct((_NC, 20480), f32),
        ),
        mesh=mesh,
        compiler_params=pltpu.CompilerParams(needs_layout_passes=False,
                                             use_tc_tiling_on_sc=False),
        scratch_types=[
            pltpu.VMEM((4 * NT,), f32),
            pltpu.VMEM((4 * NT,), f32),
            pltpu.VMEM((_EPA,), jnp.int32),
            pltpu.VMEM((_EPA,), jnp.int32),
            pltpu.VMEM((nr, _KA), jnp.int32),
            pltpu.VMEM((nr, _KA), f32),
            pltpu.VMEM((_KA,), f32),
            pltpu.VMEM_SHARED((20480,), f32),
            pltpu.SemaphoreType.DMA,
        ],
    )(as_flat, ad_flat, src, dst, z1)


def _pb_body(ytab, ztab, xtab, vtab, src_hbm, dst_hbm, w_hbm, ex_hbm, z2_hbm,
             part_hbm,
             vv, aggsp, *flat):
    c = lax.axis_index("c")
    s = lax.axis_index("s")
    SA, SB = flat[0:16], flat[16:32]
    pltpu.sync_copy(vtab.at[c], vv)
    pltpu.sync_copy(z2_hbm, aggsp.at[pl.ds(s * 1250, 1250), :])
    plsc.subcore_barrier()
    base = s * _EPB
    coff1 = c * NT
    coff2 = c * G
    lanes = lax.iota(jnp.int32, 16)

    def front_lin(ci, S, k):
        (srcv, dstv, wv, exv, ixv, iyv, izv, izs, tvv, Xv, Yv, Zv, Mv,
         semL, semG, semS) = S
        eb = base + ci * _KB
        pltpu.async_copy(src_hbm.at[pl.ds(eb, k)], srcv, semL)
        pltpu.async_copy(dst_hbm.at[pl.ds(eb, k)], dstv, semL)
        pltpu.async_copy(w_hbm.at[pl.ds(eb, k)], wv, semL)
        pltpu.async_copy(ex_hbm.at[pl.ds(eb, k)], exv, semL)

    def front(S, k):
        (srcv, dstv, wv, exv, ixv, iyv, izv, izs, tvv, Xv, Yv, Zv, Mv,
         semL, semG, semS) = S
        for r in (srcv, dstv, wv, exv):
            pltpu.make_async_copy(src_hbm.at[pl.ds(base, k)], r, semL).wait()
        for j in range(k // 16):
            sl = pl.ds(j * 16, 16)
            sv = srcv[sl]
            dv = dstv[sl]
            stv = (sv >= N).astype(jnp.int32)
            dtv = (dv >= N).astype(jnp.int32)
            ixv[sl] = coff1 + sv
            iyv[sl] = coff2 + dtv * NT + sv
            izv[sl] = coff2 + stv * NT + dv
            izs[sl] = stv * NT + dv
            tvv[sl] = 2 * stv + dtv
        pltpu.async_copy(xtab.at[ixv], Xv, semG)
        pltpu.async_copy(ytab.at[iyv], Yv, semG)
        pltpu.async_copy(ztab.at[izv], Zv, semG)

    def wait_gath(S):
        (srcv, dstv, wv, exv, ixv, iyv, izv, izs, tvv, Xv, Yv, Zv, Mv,
         semL, semG, semS) = S
        pltpu.make_async_copy(xtab.at[ixv], Xv, semG).wait()
        pltpu.make_async_copy(ytab.at[iyv], Yv, semG).wait()
        pltpu.make_async_copy(ztab.at[izv], Zv, semG).wait()

    vre = [vv[pl.ds(t * H + jj * 16, 16)] for t in range(4) for jj in range(4)]

    def compute(S, k):
        (srcv, dstv, wv, exv, ixv, iyv, izv, izs, tvv, Xv, Yv, Zv, Mv,
         semL, semG, semS) = S

        def grp(g, carry):
            tvec = tvv[pl.ds(g * 16, 16)]
            wvec = wv[pl.ds(g * 16, 16)]
            evec = exv[pl.ds(g * 16, 16)]
            bvec = evec * wvec
            for i16 in range(16):
                i = g * 16 + i16
                t_i = tvec[i16]
                a_i = evec[i16]
                b_i = bvec[i16]
                is0 = t_i == 0
                is2 = t_i == 2
                lo = t_i < 2
                for jj in range(4):
                    slc = pl.ds(jj * 16, 16)
                    v01 = jnp.where(is0, vre[jj], vre[4 + jj])
                    v23 = jnp.where(is2, vre[8 + jj], vre[12 + jj])
                    vr = jnp.where(lo, v01, v23)
                    Mv[i, slc] = (Xv[i, slc]
                                  * (a_i * (Yv[i, slc] + Zv[i, slc]) + b_i * vr))
            return carry
        lax.fori_loop(0, k // 16, grp, 0)

    def scat(S):
        (srcv, dstv, wv, exv, ixv, iyv, izv, izs, tvv, Xv, Yv, Zv, Mv,
         semL, semG, semS) = S
        pltpu.async_copy(Mv, aggsp.at[izs], semS, add=True)

    def wait_scat(S):
        (srcv, dstv, wv, exv, ixv, iyv, izv, izs, tvv, Xv, Yv, Zv, Mv,
         semL, semG, semS) = S
        pltpu.make_async_copy(Mv, aggsp.at[izs], semS).wait()

    front_lin(0, SA, _KB)
    front(SA, _KB)
    front_lin(1, SB, _KB)
    pltpu.async_copy(z2_hbm.at[pl.ds(0, _KB), :], SB[12], SB[15])

    def pairs(ci2, carry):
        ca = 2 * ci2
        wait_gath(SA)
        wait_scat(SB)
        front(SB, _KB)
        compute(SA, _KB)
        scat(SA)
        front_lin(ca + 2, SA, _KB)
        wait_gath(SB)
        wait_scat(SA)
        front(SA, _KB)
        compute(SB, _KB)
        scat(SB)
        front_lin(ca + 3, SB, _KB)
        return carry
    lax.fori_loop(0, (_NFB - 2) // 2, pairs, 0)
    wait_gath(SA)
    wait_scat(SB)
    front(SB, _KB)
    compute(SA, _KB)
    scat(SA)
    wait_gath(SB)
    wait_scat(SA)
    compute(SB, _KB)
    scat(SB)
    wait_scat(SB)
    plsc.subcore_barrier()
    pltpu.sync_copy(aggsp.at[pl.ds(s * 1250, 1250), :],
                    part_hbm.at[c, pl.ds(s * 1250, 1250), :])


def _sc_pass_b(ytab, ztab, xtab, vtab, src, dst, w, ex, z2):
    f32, i32 = jnp.float32, jnp.int32
    mesh = plsc.VectorSubcoreMesh(core_axis_name="c", subcore_axis_name="s")

    def bufs(k):
        return [
            pltpu.VMEM((k,), i32),
            pltpu.VMEM((k,), i32),
            pltpu.VMEM((k,), f32),
            pltpu.VMEM((k,), f32),
            pltpu.VMEM((k,), i32),
            pltpu.VMEM((k,), i32),
            pltpu.VMEM((k,), i32),
            pltpu.VMEM((k,), i32),
            pltpu.VMEM((k,), i32),
            pltpu.VMEM((k, H), f32),
            pltpu.VMEM((k, H), f32),
            pltpu.VMEM((k, H), f32),
            pltpu.VMEM((k, H), f32),
            pltpu.SemaphoreType.DMA,
            pltpu.SemaphoreType.DMA,
            pltpu.SemaphoreType.DMA,
        ]
    return pl.kernel(
        _pb_body,
        out_type=jax.ShapeDtypeStruct((_NC, GP, H), f32),
        mesh=mesh,
        compiler_params=pltpu.CompilerParams(needs_layout_passes=False,
                                             use_tc_tiling_on_sc=False),
        scratch_types=(
            [pltpu.VMEM((4 * H,), f32),
             pltpu.VMEM_SHARED((GP, H), f32)]
            + bufs(_KB) + bufs(_KB)
        ),
    )(ytab, ztab, xtab, vtab, src, dst, w, ex, z2)


def _post_body(p_ref, den_ref, x_ref, wagg1_ref, wagg2_ref, bagg_ref,
               wks_ref, bks_ref, qs_ref, wkt_ref, bkt_ref, qt_ref, out_ref):
    outs = []
    for t, (st, dn) in enumerate(((0, 0), (0, 1), (1, 0), (1, 1))):
        r0 = st * NT + dn * N
        ag = jnp.concatenate(
            [p_ref[0, pl.ds(r0, N), :], p_ref[1, pl.ds(r0, N), :]], axis=1)
        den = (den_ref[0, pl.ds(r0, N), 0:1]
               + den_ref[1, pl.ds(r0, N), 0:1] + 1e-16)
        ag = ag / den
        xd = x_ref[pl.ds(dn * N, N), :]
        o = (jnp.dot(ag, wagg1_ref[t], preferred_element_type=jnp.float32)
             + jnp.dot(xd, wagg2_ref[t], preferred_element_type=jnp.float32)
             + bagg_ref[pl.ds(t, 1), :])
        outs.append(jax.nn.relu(o))

    def group(o0, o1, wk_ref, bk_ref, q_ref, row0):
        m0 = jnp.mean(jnp.tanh(
            jnp.dot(o0, wk_ref[...], preferred_element_type=jnp.float32)
            + bk_ref[...]), axis=0, keepdims=True)
        m1 = jnp.mean(jnp.tanh(
            jnp.dot(o1, wk_ref[...], preferred_element_type=jnp.float32)
            + bk_ref[...]), axis=0, keepdims=True)
        s0 = jnp.sum(q_ref[...] * m0, axis=1, keepdims=True)
        s1 = jnp.sum(q_ref[...] * m1, axis=1, keepdims=True)
        sm = jnp.maximum(s0, s1)
        e0 = jnp.exp(s0 - sm)
        e1 = jnp.exp(s1 - sm)
        tot = e0 + e1
        out_ref[pl.ds(row0, N), :] = (e0 / tot) * o0 + (e1 / tot) * o1

    group(outs[0], outs[2], wks_ref, bks_ref, qs_ref, 0)
    group(outs[3], outs[1], wkt_ref, bkt_ref, qt_ref, N)


def _tc_post(p, den, x, wagg1, wagg2, bagg, wks, bks, qs, wkt, bkt, qt):
    return pl.pallas_call(
        _post_body,
        compiler_params=pltpu.CompilerParams(vmem_limit_bytes=100 * 1024 * 1024),
        out_shape=jax.ShapeDtypeStruct((NT, C), jnp.float32),
    )(p, den, x, wagg1, wagg2, bagg, wks, bks, qs, wkt, bkt, qt)


def kernel(x, edge_index, edge_weight, params):
    f32 = jnp.float32
    wj4 = jnp.stack([params['W_att_' + et][0:C] for et in ETS])
    wi4 = jnp.stack([params['W_att_' + et][C:2 * C] for et in ETS])
    w34 = jnp.stack([params['W_att_' + et][2 * C:3 * C] for et in ETS])
    wep4 = jnp.stack([params['W_ep_' + et][0] for et in ETS])
    bep4 = jnp.stack([params['b_ep_' + et] for et in ETS])
    batt4 = jnp.stack([params['b_att_' + et] for et in ETS])
    lsT = jnp.stack([params['lsrc_' + et] for et in ETS], axis=1)
    ldT = jnp.stack([params['ldst_' + et] for et in ETS], axis=1)

    y, z, as_, ad_, v = _tc_precompute(
        x, wj4, wi4, w34, wep4, bep4, batt4, lsT, ldT)
    xh = jnp.stack([x[:, :H], x[:, H:]])

    src = edge_index[0]
    dst = edge_index[1]
    z1 = jnp.zeros((1280,), f32)
    ex, den = _sc_pass_a(as_.reshape(-1), ad_.reshape(-1), src, dst, z1)
    ex = ex.reshape(E)
    den = den.reshape(_NC, 20480, 1)

    z2 = jnp.zeros((1250, H), f32)
    part = _sc_pass_b(y.reshape(2 * G, H), z.reshape(2 * G, H),
                      xh.reshape(2 * NT, H), v.reshape(2, 4 * H),
                      src, dst, edge_weight, ex, z2)

    wagg1 = jnp.stack([params['W_agg_' + et][:C] for et in ETS])
    wagg2 = jnp.stack([params['W_agg_' + et][C:] for et in ETS])
    bagg = jnp.stack([params['b_agg_' + et] for et in ETS])
    return _tc_post(part, den, x, wagg1, wagg2, bagg,
                    params['Wk_s'], params['bk_s'].reshape(1, C), params['q_s'],
                    params['Wk_t'], params['bk_t'].reshape(1, C), params['q_t'])
